# baseline (speedup 1.0000x reference)
import jax
import jax.numpy as jnp
from jax import lax
from jax.experimental import pallas as pl
from jax.experimental.pallas import tpu as pltpu
from jax.experimental.pallas import tpu_sc as plsc

_E = 64
_F = 8192
_EPS = 1e-06
_NC, _NS = 2, 16
_NW = _NC * _NS
_TOKENS = 4 * 8192
_TPW = _TOKENS // _NW
_GRP = _TPW // 16
_DEPTH = 8


def _cp_body(e_ref, g_ref, eo_ref, go_ref):
    eo_ref[...] = e_ref[0].T
    go_ref[...] = g_ref[0].T


def _sc_body(e_hbm, g_hbm, f_hbm, fec_hbm, counts_hbm, imp_hbm,
             e_v, g_v, f_v, hist, imp, acc, sem):
    c = lax.axis_index("c")
    s = lax.axis_index("s")
    wid = c * _NS + s

    d1 = pltpu.async_copy(e_hbm.at[pl.ds(0, 8), pl.ds(wid * _TPW, _TPW)], e_v, sem)
    d2 = pltpu.async_copy(g_hbm.at[pl.ds(0, 8), pl.ds(wid * _TPW, _TPW)], g_v, sem)
    d3 = pltpu.async_copy(f_hbm.at[pl.ds(wid * _GRP, _GRP)], f_v, sem)
    rows = _F // _NS
    d4 = pltpu.async_copy(fec_hbm.at[pl.ds(s * rows, rows)],
                          acc.at[pl.ds(s * rows, rows)], sem)

    z16 = jnp.zeros((16,), jnp.float32)

    @pl.loop(0, _GRP)
    def _z(i):
        imp[i, :] = z16
        for r in range(16):
            for j in range(4):
                hist[i, r, pl.ds(j * 16, 16)] = z16

    d1.wait()
    d2.wait()
    d3.wait()
    d4.wait()
    plsc.subcore_barrier()

    ones16 = jnp.ones((16,), jnp.float32)
    lio8 = lax.iota(jnp.int32, 16) * 8
    liota = lax.iota(jnp.int32, 16)

    @pl.loop(0, _GRP)
    def _grp(g):
        for k in range(8):
            ev = e_v[k, pl.ds(g * 16, 16)]
            plsc.addupdate_scatter(hist.at[g], [liota, ev], ones16)
            gv = g_v[k, pl.ds(g * 16, 16)]
            plsc.addupdate_scatter(imp, [ev, liota], gv)
        pltpu.async_copy(hist.at[g], acc.at[f_v.at[g]], sem, add=True)
        @pl.when(g >= _DEPTH)
        def _():
            pltpu.make_async_copy(fec_hbm.at[pl.ds(0, 16)], hist.at[0], sem).wait()

    @pl.loop(0, _DEPTH)
    def _drain(i):
        pltpu.make_async_copy(fec_hbm.at[pl.ds(0, 16)], hist.at[0], sem).wait()

    pltpu.sync_copy(imp, imp_hbm.at[wid])

    plsc.subcore_barrier()
    pltpu.sync_copy(acc.at[pl.ds(s * rows, rows)],
                    counts_hbm.at[c, pl.ds(s * rows, rows)])


def _tc_body(counts_ref, fec_ref, imp_ref, out_ref):
    fec = counts_ref[0] + counts_ref[1] - fec_ref[...]
    ssum = jnp.sum(fec, axis=1, keepdims=True)
    p = fec / (ssum + _EPS)
    spec = -jnp.sum(p * jnp.log(p + _EPS)) / (_F * _E)

    imp2d = jnp.sum(imp_ref[...], axis=0)
    impv = jnp.sum(imp2d, axis=1, keepdims=True)
    m = jnp.sum(impv) / _E
    var = jnp.sum((impv - m) ** 2) / (_E - 1)
    balance = jnp.sqrt(var) / (m + _EPS)
    loss = balance + (1.0 - spec)

    lane = lax.broadcasted_iota(jnp.int32, (1, 128), 1)
    out_ref[...] = jnp.where(lane == 0, loss,
                             jnp.where(lane == 1, balance, spec))


def kernel(gates, expert_indices, feature_indices, feature_expert_counts):
    e_c, g_c = pl.pallas_call(
        _cp_body,
        grid=(4, 8),
        in_specs=[
            pl.BlockSpec((1, 1024, 8), lambda b, i: (b, i, 0)),
            pl.BlockSpec((1, 1024, 8), lambda b, i: (b, i, 0)),
        ],
        out_specs=[
            pl.BlockSpec((8, 1024), lambda b, i: (0, b * 8 + i)),
            pl.BlockSpec((8, 1024), lambda b, i: (0, b * 8 + i)),
        ],
        out_shape=[
            jax.ShapeDtypeStruct((8, _TOKENS), jnp.int32),
            jax.ShapeDtypeStruct((8, _TOKENS), jnp.float32),
        ],
    )(expert_indices.astype(jnp.int32), gates)
    e_flat = e_c
    g_flat = g_c
    f2d = feature_indices.reshape(-1, 16).astype(jnp.int32)
    fec = feature_expert_counts

    mesh = plsc.VectorSubcoreMesh(core_axis_name="c", subcore_axis_name="s",
                                  num_cores=_NC, num_subcores=_NS)
    sc_call = pl.kernel(
        _sc_body,
        out_type=[
            jax.ShapeDtypeStruct((_NC, _F, _E), jnp.float32),
            jax.ShapeDtypeStruct((_NW, _E, 16), jnp.float32),
        ],
        mesh=mesh,
        scratch_types=[
            pltpu.VMEM((8, _TPW), jnp.int32),
            pltpu.VMEM((8, _TPW), jnp.float32),
            pltpu.VMEM((_GRP, 16), jnp.int32),
            pltpu.VMEM((_GRP, 16, _E), jnp.float32),
            pltpu.VMEM((_E, 16), jnp.float32),
            pltpu.VMEM_SHARED((_F, _E), jnp.float32),
            pltpu.SemaphoreType.DMA,
        ],
        compiler_params=pltpu.CompilerParams(needs_layout_passes=False,
                                             use_tc_tiling_on_sc=False),
    )
    counts2, imp32 = sc_call(e_flat, g_flat, f2d, fec)

    out = pl.pallas_call(
        _tc_body,
        out_shape=jax.ShapeDtypeStruct((1, 128), jnp.float32),
    )(counts2, fec, imp32)

    return out[0, 0], out[0, 1], out[0, 2]


# speedup vs baseline: 1.0186x; 1.0186x over previous
import jax
import jax.numpy as jnp
from jax import lax
from jax.experimental import pallas as pl
from jax.experimental.pallas import tpu as pltpu
from jax.experimental.pallas import tpu_sc as plsc

_E = 64
_F = 8192
_EPS = 1e-06
_NC, _NS = 2, 16
_NW = _NC * _NS
_TOKENS = 4 * 8192
_TPW = _TOKENS // _NW
_GRP = _TPW // 16
_DEPTH = 8


def _cp_body(e_ref, g_ref, eo_ref, go_ref):
    j16 = lax.broadcasted_iota(jnp.int32, (1, 16, 128), 1)
    lane = lax.broadcasted_iota(jnp.int32, (1, 16, 128), 2)
    m = (lane // 8 == j16).astype(jnp.float32)
    b = (lax.broadcasted_iota(jnp.int32, (8, 128), 1) % 8 ==
         lax.broadcasted_iota(jnp.int32, (8, 128), 0)).astype(jnp.float32)
    ye = jnp.dot(e_ref[0].astype(jnp.float32), b,
                 preferred_element_type=jnp.float32)
    eo_ref[...] = jnp.sum(ye.reshape(64, 16, 128) * m, axis=1).astype(jnp.int32)
    yg = jnp.dot(g_ref[0], b, preferred_element_type=jnp.float32)
    go_ref[...] = jnp.sum(yg.reshape(64, 16, 128) * m, axis=1)


def _sc_body(e_hbm, g_hbm, f_hbm, fec_hbm, counts_hbm, imp_hbm,
             e_v, g_v, f_v, hist, imp, acc, sem):
    c = lax.axis_index("c")
    s = lax.axis_index("s")
    wid = c * _NS + s

    d1 = pltpu.async_copy(e_hbm.at[pl.ds(wid * 64, 64)], e_v, sem)
    d2 = pltpu.async_copy(g_hbm.at[pl.ds(wid * 64, 64)], g_v, sem)
    d3 = pltpu.async_copy(f_hbm.at[pl.ds(wid * _GRP, _GRP)], f_v, sem)
    rows = _F // _NS
    d4 = pltpu.async_copy(fec_hbm.at[pl.ds(s * rows, rows)],
                          acc.at[pl.ds(s * rows, rows)], sem)

    z16 = jnp.zeros((16,), jnp.float32)

    @pl.loop(0, _GRP)
    def _z(i):
        imp[i, :] = z16
        for r in range(16):
            for j in range(4):
                hist[i, r, pl.ds(j * 16, 16)] = z16

    d1.wait()
    d2.wait()
    d3.wait()
    d4.wait()
    plsc.subcore_barrier()

    ones16 = jnp.ones((16,), jnp.float32)
    lio8 = lax.iota(jnp.int32, 16) * 8
    liota = lax.iota(jnp.int32, 16)

    @pl.loop(0, _GRP)
    def _grp(g):
        base = g * 128
        for k in range(8):
            idx = lio8 + (base + k)
            row = lax.shift_right_logical(idx, 7)
            col = lax.bitwise_and(idx, 127)
            ev = plsc.load_gather(e_v, [row, col])
            plsc.addupdate_scatter(hist.at[g], [liota, ev], ones16)
            gv = plsc.load_gather(g_v, [row, col])
            plsc.addupdate_scatter(imp, [ev, liota], gv)
        pltpu.async_copy(hist.at[g], acc.at[f_v.at[g]], sem, add=True)
        @pl.when(g >= _DEPTH)
        def _():
            pltpu.make_async_copy(fec_hbm.at[pl.ds(0, 16)], hist.at[0], sem).wait()

    @pl.loop(0, _DEPTH)
    def _drain(i):
        pltpu.make_async_copy(fec_hbm.at[pl.ds(0, 16)], hist.at[0], sem).wait()

    pltpu.sync_copy(imp, imp_hbm.at[wid])

    plsc.subcore_barrier()
    pltpu.sync_copy(acc.at[pl.ds(s * rows, rows)],
                    counts_hbm.at[c, pl.ds(s * rows, rows)])


def _tc_body(counts_ref, fec_ref, imp_ref, out_ref):
    fec = counts_ref[0] + counts_ref[1] - fec_ref[...]
    ssum = jnp.sum(fec, axis=1, keepdims=True)
    p = fec / (ssum + _EPS)
    spec = -jnp.sum(p * jnp.log(p + _EPS)) / (_F * _E)

    imp2d = jnp.sum(imp_ref[...], axis=0)
    impv = jnp.sum(imp2d, axis=1, keepdims=True)
    m = jnp.sum(impv) / _E
    var = jnp.sum((impv - m) ** 2) / (_E - 1)
    balance = jnp.sqrt(var) / (m + _EPS)
    loss = balance + (1.0 - spec)

    lane = lax.broadcasted_iota(jnp.int32, (1, 128), 1)
    out_ref[...] = jnp.where(lane == 0, loss,
                             jnp.where(lane == 1, balance, spec))


def kernel(gates, expert_indices, feature_indices, feature_expert_counts):
    e_flat, g_flat = pl.pallas_call(
        _cp_body,
        grid=(4, 8),
        in_specs=[
            pl.BlockSpec((1, 1024, 8), lambda b, i: (b, i, 0)),
            pl.BlockSpec((1, 1024, 8), lambda b, i: (b, i, 0)),
        ],
        out_specs=[
            pl.BlockSpec((64, 128), lambda b, i: (b * 8 + i, 0)),
            pl.BlockSpec((64, 128), lambda b, i: (b * 8 + i, 0)),
        ],
        out_shape=[
            jax.ShapeDtypeStruct((2048, 128), jnp.int32),
            jax.ShapeDtypeStruct((2048, 128), jnp.float32),
        ],
    )(expert_indices.astype(jnp.int32), gates)
    f2d = feature_indices.reshape(-1, 16).astype(jnp.int32)
    fec = feature_expert_counts

    mesh = plsc.VectorSubcoreMesh(core_axis_name="c", subcore_axis_name="s",
                                  num_cores=_NC, num_subcores=_NS)
    sc_call = pl.kernel(
        _sc_body,
        out_type=[
            jax.ShapeDtypeStruct((_NC, _F, _E), jnp.float32),
            jax.ShapeDtypeStruct((_NW, _E, 16), jnp.float32),
        ],
        mesh=mesh,
        scratch_types=[
            pltpu.VMEM((64, 128), jnp.int32),
            pltpu.VMEM((64, 128), jnp.float32),
            pltpu.VMEM((_GRP, 16), jnp.int32),
            pltpu.VMEM((_GRP, 16, _E), jnp.float32),
            pltpu.VMEM((_E, 16), jnp.float32),
            pltpu.VMEM_SHARED((_F, _E), jnp.float32),
            pltpu.SemaphoreType.DMA,
        ],
        compiler_params=pltpu.CompilerParams(needs_layout_passes=False,
                                             use_tc_tiling_on_sc=False),
    )
    counts2, imp32 = sc_call(e_flat, g_flat, f2d, fec)

    out = pl.pallas_call(
        _tc_body,
        out_shape=jax.ShapeDtypeStruct((1, 128), jnp.float32),
    )(counts2, fec, imp32)

    return out[0, 0], out[0, 1], out[0, 2]


# speedup vs baseline: 1.0561x; 1.0368x over previous
import jax
import jax.numpy as jnp
from jax import lax
from jax.experimental import pallas as pl
from jax.experimental.pallas import tpu as pltpu
from jax.experimental.pallas import tpu_sc as plsc

_E = 64
_F = 8192
_EPS = 1e-06
_NC, _NS = 2, 16
_NW = _NC * _NS
_TOKENS = 4 * 8192
_TPW = _TOKENS // _NW
_GRP = _TPW // 16
_DEPTH = 8


def _sc_body(e_hbm, g_hbm, f_hbm, fec_hbm, counts_hbm, imp_hbm,
             e_v, g_v, f_v, hist, imp, acc, sem):
    c = lax.axis_index("c")
    s = lax.axis_index("s")
    wid = c * _NS + s

    d1 = pltpu.async_copy(e_hbm.at[pl.ds(wid * _TPW * 8, _TPW * 8)], e_v, sem)
    d2 = pltpu.async_copy(g_hbm.at[pl.ds(wid * _TPW * 8, _TPW * 8)], g_v, sem)
    d3 = pltpu.async_copy(f_hbm.at[pl.ds(wid * _GRP, _GRP)], f_v, sem)
    rows = _F // _NS
    d4 = pltpu.async_copy(fec_hbm.at[pl.ds(s * rows, rows)],
                          acc.at[pl.ds(s * rows, rows)], sem)

    z16 = jnp.zeros((16,), jnp.float32)

    @pl.loop(0, _GRP)
    def _z(i):
        imp[i, :] = z16
        for r in range(16):
            for j in range(4):
                hist[i, r, pl.ds(j * 16, 16)] = z16

    d1.wait()
    d2.wait()
    d3.wait()
    d4.wait()
    plsc.subcore_barrier()

    ones16 = jnp.ones((16,), jnp.float32)
    lio8 = lax.iota(jnp.int32, 16) * 8
    liota = lax.iota(jnp.int32, 16)

    @pl.loop(0, _GRP)
    def _grp(g):
        base = g * 128
        for k in range(8):
            idx = lio8 + (base + k)
            ev = plsc.load_gather(e_v, [idx])
            plsc.addupdate_scatter(hist.at[g], [liota, ev], ones16)
            gv = plsc.load_gather(g_v, [idx])
            plsc.addupdate_scatter(imp, [ev, liota], gv)
        pltpu.async_copy(hist.at[g], acc.at[f_v.at[g]], sem, add=True)
        @pl.when(g >= _DEPTH)
        def _():
            pltpu.make_async_copy(fec_hbm.at[pl.ds(0, 16)], hist.at[0], sem).wait()

    @pl.loop(0, _DEPTH)
    def _drain(i):
        pltpu.make_async_copy(fec_hbm.at[pl.ds(0, 16)], hist.at[0], sem).wait()

    pltpu.sync_copy(imp, imp_hbm.at[wid])

    plsc.subcore_barrier()
    pltpu.sync_copy(acc.at[pl.ds(s * rows, rows)],
                    counts_hbm.at[c, pl.ds(s * rows, rows)])


def _tc_body(counts_ref, fec_ref, imp_ref, out_ref):
    fec = counts_ref[0] + counts_ref[1] - fec_ref[...]
    ssum = jnp.sum(fec, axis=1, keepdims=True)
    p = fec / (ssum + _EPS)
    spec = -jnp.sum(p * jnp.log(p + _EPS)) / (_F * _E)

    imp2d = jnp.sum(imp_ref[...], axis=0)
    impv = jnp.sum(imp2d, axis=1, keepdims=True)
    m = jnp.sum(impv) / _E
    var = jnp.sum((impv - m) ** 2) / (_E - 1)
    balance = jnp.sqrt(var) / (m + _EPS)
    loss = balance + (1.0 - spec)

    lane = lax.broadcasted_iota(jnp.int32, (1, 128), 1)
    out_ref[...] = jnp.where(lane == 0, loss,
                             jnp.where(lane == 1, balance, spec))


def kernel(gates, expert_indices, feature_indices, feature_expert_counts):
    e_flat = expert_indices.reshape(-1).astype(jnp.int32)
    g_flat = gates.reshape(-1)
    f2d = feature_indices.reshape(-1, 16).astype(jnp.int32)
    fec = feature_expert_counts

    mesh = plsc.VectorSubcoreMesh(core_axis_name="c", subcore_axis_name="s",
                                  num_cores=_NC, num_subcores=_NS)
    sc_call = pl.kernel(
        _sc_body,
        out_type=[
            jax.ShapeDtypeStruct((_NC, _F, _E), jnp.float32),
            jax.ShapeDtypeStruct((_NW, _E, 16), jnp.float32),
        ],
        mesh=mesh,
        scratch_types=[
            pltpu.VMEM((_TPW * 8,), jnp.int32),
            pltpu.VMEM((_TPW * 8,), jnp.float32),
            pltpu.VMEM((_GRP, 16), jnp.int32),
            pltpu.VMEM((_GRP, 16, _E), jnp.float32),
            pltpu.VMEM((_E, 16), jnp.float32),
            pltpu.VMEM_SHARED((_F, _E), jnp.float32),
            pltpu.SemaphoreType.DMA,
        ],
        compiler_params=pltpu.CompilerParams(needs_layout_passes=False,
                                             use_tc_tiling_on_sc=False),
    )
    counts2, imp32 = sc_call(e_flat, g_flat, f2d, fec)

    out = pl.pallas_call(
        _tc_body,
        out_shape=jax.ShapeDtypeStruct((1, 128), jnp.float32),
    )(counts2, fec, imp32)

    return out[0, 0], out[0, 1], out[0, 2]
